# baseline (speedup 1.0000x reference)
import jax
import jax.numpy as jnp
from jax.experimental import pallas as pl

B = 1024
C = 3
HW = 224 * 224
E = 64
R = B * C

BR = 192
BC = 12544
NR = R // BR
NC = HW // BC


def _pool_kernel(x_ref, sums_ref):
    j = pl.program_id(1)
    s = jnp.sum(x_ref[...], axis=1, keepdims=True)

    @pl.when(j == 0)
    def _():
        sums_ref[...] = s

    @pl.when(j != 0)
    def _():
        sums_ref[...] += s


def _routing_kernel(p_ref, wt_ref, b_ref, gates_ref, idx_ref):
    n = jnp.float32(HW)
    p0 = (p_ref[:, 0:1] / n).astype(jnp.bfloat16).astype(jnp.float32)
    p1 = (p_ref[:, 1:2] / n).astype(jnp.bfloat16).astype(jnp.float32)
    p2 = (p_ref[:, 2:3] / n).astype(jnp.bfloat16).astype(jnp.float32)
    wb = wt_ref[...].astype(jnp.bfloat16).astype(jnp.float32)
    logits = (p0 * wb[0:1, :] + p1 * wb[1:2, :]) + p2 * wb[2:3, :]
    logits = logits + b_ref[...]

    iota = jax.lax.broadcasted_iota(jnp.int32, (B, E), 1)
    m0 = jnp.max(logits, axis=1, keepdims=True)
    idx0 = jnp.min(jnp.where(logits == m0, iota, E), axis=1, keepdims=True)
    masked = jnp.where(iota == idx0, jnp.finfo(jnp.float32).min, logits)
    m1 = jnp.max(masked, axis=1, keepdims=True)
    idx1 = jnp.min(jnp.where(masked == m1, iota, E), axis=1, keepdims=True)

    e1 = jnp.exp(m1 - m0)
    denom = 1.0 + e1
    g0 = 1.0 / denom
    g1 = e1 / denom

    gates_ref[...] = jnp.where(iota == idx0, g0,
                               jnp.where(iota == idx1, g1, 0.0))
    idx_ref[...] = jnp.concatenate([idx0, idx1], axis=1)


_pool = pl.pallas_call(
    _pool_kernel,
    grid=(NR, NC),
    in_specs=[pl.BlockSpec((BR, BC), lambda i, j: (i, j))],
    out_specs=pl.BlockSpec((BR, 1), lambda i, j: (i, 0)),
    out_shape=jax.ShapeDtypeStruct((R, 1), jnp.float32),
)

_route = pl.pallas_call(
    _routing_kernel,
    in_specs=[
        pl.BlockSpec((B, C), lambda: (0, 0)),
        pl.BlockSpec((C, E), lambda: (0, 0)),
        pl.BlockSpec((1, E), lambda: (0, 0)),
    ],
    out_specs=[
        pl.BlockSpec((B, E), lambda: (0, 0)),
        pl.BlockSpec((B, 2), lambda: (0, 0)),
    ],
    out_shape=[
        jax.ShapeDtypeStruct((B, E), jnp.float32),
        jax.ShapeDtypeStruct((B, 2), jnp.int32),
    ],
)


def kernel(x, W, b):
    sums = _pool(x.reshape(R, HW))
    gates, idx = _route(sums.reshape(B, C), W.T, b.reshape(1, E))
    return (gates, idx)


# speedup vs baseline: 1.6015x; 1.6015x over previous
import jax
import jax.numpy as jnp
from jax.experimental import pallas as pl
from jax.experimental.pallas import tpu as pltpu

B = 1024
C = 3
H = 224
W_DIM = 224
HW = H * W_DIM
E = 64

BB = 16
NB = B // BB


def _router_kernel(x_ref, wt_ref, b_ref, gates_ref, idx_ref, acc_ref):
    i = pl.program_id(0)
    s = jnp.sum(x_ref[...], axis=(2, 3))
    acc_ref[pl.ds(i * BB, BB), :] = s

    @pl.when(i == NB - 1)
    def _finish():
        n = jnp.float32(HW)
        p0 = (acc_ref[:, 0:1] / n).astype(jnp.bfloat16).astype(jnp.float32)
        p1 = (acc_ref[:, 1:2] / n).astype(jnp.bfloat16).astype(jnp.float32)
        p2 = (acc_ref[:, 2:3] / n).astype(jnp.bfloat16).astype(jnp.float32)
        wb = wt_ref[...].astype(jnp.bfloat16).astype(jnp.float32)
        logits = (p0 * wb[0:1, :] + p1 * wb[1:2, :]) + p2 * wb[2:3, :]
        logits = logits + b_ref[...]

        iota = jax.lax.broadcasted_iota(jnp.int32, (B, E), 1)
        m0 = jnp.max(logits, axis=1, keepdims=True)
        idx0 = jnp.min(jnp.where(logits == m0, iota, E), axis=1,
                       keepdims=True)
        masked = jnp.where(iota == idx0, jnp.finfo(jnp.float32).min, logits)
        m1 = jnp.max(masked, axis=1, keepdims=True)
        idx1 = jnp.min(jnp.where(masked == m1, iota, E), axis=1,
                       keepdims=True)

        e1 = jnp.exp(m1 - m0)
        denom = 1.0 + e1
        g0 = 1.0 / denom
        g1 = e1 / denom

        gates_ref[...] = jnp.where(iota == idx0, g0,
                                   jnp.where(iota == idx1, g1, 0.0))
        idx_ref[...] = jnp.concatenate([idx0, idx1], axis=1)


_router = pl.pallas_call(
    _router_kernel,
    grid=(NB,),
    in_specs=[
        pl.BlockSpec((BB, C, H, W_DIM), lambda i: (i, 0, 0, 0)),
        pl.BlockSpec((C, E), lambda i: (0, 0)),
        pl.BlockSpec((1, E), lambda i: (0, 0)),
    ],
    out_specs=[
        pl.BlockSpec((B, E), lambda i: (0, 0)),
        pl.BlockSpec((B, 2), lambda i: (0, 0)),
    ],
    out_shape=[
        jax.ShapeDtypeStruct((B, E), jnp.float32),
        jax.ShapeDtypeStruct((B, 2), jnp.int32),
    ],
    scratch_shapes=[pltpu.VMEM((B, C), jnp.float32)],
)


def kernel(x, W, b):
    gates, idx = _router(x, W.T, b.reshape(1, E))
    return (gates, idx)


# speedup vs baseline: 7.3349x; 4.5800x over previous
import jax
import jax.numpy as jnp
from jax.experimental import pallas as pl
from jax.experimental.pallas import tpu as pltpu

B = 1024
C = 3
H = 224
W_DIM = 224
HW = H * W_DIM
E = 64

BH = 16
NH = H // BH


def _router_kernel(x_ref, w_ref, b_ref, gates_ref, idx_ref, acc_ref):
    c = pl.program_id(0)
    j = pl.program_id(1)
    s = jnp.sum(x_ref[0], axis=(0, 1), keepdims=True)[0]

    @pl.when(j == 0)
    def _():
        acc_ref[c] = s

    @pl.when(j != 0)
    def _():
        acc_ref[c] += s

    @pl.when((c == C - 1) & (j == NH - 1))
    def _finish():
        n = jnp.float32(HW)
        p0 = (acc_ref[0] / n).astype(jnp.bfloat16).astype(jnp.float32)
        p1 = (acc_ref[1] / n).astype(jnp.bfloat16).astype(jnp.float32)
        p2 = (acc_ref[2] / n).astype(jnp.bfloat16).astype(jnp.float32)
        wb = w_ref[...].astype(jnp.bfloat16).astype(jnp.float32)
        logits = (wb[:, 0:1] * p0 + wb[:, 1:2] * p1) + wb[:, 2:3] * p2
        logits = logits + b_ref[...]

        iota = jax.lax.broadcasted_iota(jnp.int32, (E, B), 0)
        m0 = jnp.max(logits, axis=0, keepdims=True)
        idx0 = jnp.min(jnp.where(logits == m0, iota, E), axis=0,
                       keepdims=True)
        masked = jnp.where(iota == idx0, jnp.finfo(jnp.float32).min, logits)
        m1 = jnp.max(masked, axis=0, keepdims=True)
        idx1 = jnp.min(jnp.where(masked == m1, iota, E), axis=0,
                       keepdims=True)

        e1 = jnp.exp(m1 - m0)
        denom = 1.0 + e1
        g0 = 1.0 / denom
        g1 = e1 / denom

        gates_ref[...] = jnp.where(iota == idx0, g0,
                                   jnp.where(iota == idx1, g1, 0.0))
        idx_ref[...] = jnp.concatenate([idx0, idx1], axis=0)


_router = pl.pallas_call(
    _router_kernel,
    grid=(C, NH),
    in_specs=[
        pl.BlockSpec((1, BH, W_DIM, B), lambda c, j: (c, j, 0, 0)),
        pl.BlockSpec((E, C), lambda c, j: (0, 0)),
        pl.BlockSpec((E, 1), lambda c, j: (0, 0)),
    ],
    out_specs=[
        pl.BlockSpec((E, B), lambda c, j: (0, 0)),
        pl.BlockSpec((2, B), lambda c, j: (0, 0)),
    ],
    out_shape=[
        jax.ShapeDtypeStruct((E, B), jnp.float32),
        jax.ShapeDtypeStruct((2, B), jnp.int32),
    ],
    scratch_shapes=[pltpu.VMEM((C, 1, B), jnp.float32)],
)


def kernel(x, W, b):
    xt = jnp.transpose(x, (1, 2, 3, 0))
    gates_t, idx_t = _router(xt, W, b.reshape(E, 1))
    return (gates_t.T, idx_t.T)
